# baseline (speedup 1.0000x reference)
import functools

import numpy as np
import jax
import jax.numpy as jnp
from jax import lax
from jax.experimental import pallas as pl
from jax.experimental.pallas import tpu as pltpu, tpu_sc as plsc

VOCAB = 1000000
D = 64
B = 32
S = 2048

NC, NS, L = 2, 16, 16
NW = NC * NS
SPW = S // NW
BCHUNK = 8
NCHUNK = B // BCHUNK


def _positional_encoding_np(position, d_model):
    angle_rads = 1 / np.power(10000, 2 * (np.arange(d_model)[np.newaxis, :] // 2) / np.float32(d_model))
    angle_rads = np.arange(position)[:, np.newaxis] * angle_rads
    angle_rads[:, 0::2] = np.sin(angle_rads[:, 0::2])
    angle_rads[:, 1::2] = np.cos(angle_rads[:, 1::2])
    return angle_rads.astype(np.float32)


_POS = _positional_encoding_np(S, D)


def _body(x_hbm, pos_hbm, table_hbm, out_hbm, idx_v, pos_v, rows_v, gsem, wsem):
    wid = lax.axis_index("s") * NC + lax.axis_index("c")
    sbase = wid * SPW

    handles = [pltpu.async_copy(x_hbm.at[pl.ds(b * S + sbase, SPW)],
                                idx_v.at[b], gsem) for b in range(B)]
    pltpu.sync_copy(pos_hbm.at[pl.ds(sbase, SPW), :], pos_v)
    for h in handles:
        h.wait()

    def fire_gathers(cc, buf):
        return [pltpu.async_copy(table_hbm.at[idx_v.at[cc * BCHUNK + bb]],
                                 rows_v.at[buf, bb], gsem)
                for bb in range(BCHUNK)]

    gh = fire_gathers(0, 0)
    wh = []
    for cc in range(NCHUNK):
        cur = cc % 2
        for h in gh:
            h.wait()
        if cc + 1 < NCHUNK:
            for h in wh:
                h.wait()
            wh = []
            gh = fire_gathers(cc + 1, 1 - cur)

        def add_pos(j, carry):
            p = [pos_v[j, pl.ds(c * L, L)] for c in range(D // L)]
            for bb in range(BCHUNK):
                for c in range(D // L):
                    rows_v[cur, bb, j, pl.ds(c * L, L)] = (
                        rows_v[cur, bb, j, pl.ds(c * L, L)] + p[c])
            return carry
        lax.fori_loop(0, SPW, add_pos, 0)

        for h in wh:
            h.wait()
        wh = [pltpu.async_copy(rows_v.at[cur, bb],
                               out_hbm.at[pl.ds((cc * BCHUNK + bb) * S + sbase,
                                                SPW), pl.ds(0, D)], wsem)
              for bb in range(BCHUNK)]
    for h in wh:
        h.wait()


@jax.jit
def _run(x, pos, table):
    t3 = lax.transpose(lax.optimization_barrier(table.T), (1, 0))
    mesh = plsc.VectorSubcoreMesh(core_axis_name="c", subcore_axis_name="s",
                                  num_cores=NC, num_subcores=NS)
    f = functools.partial(
        pl.kernel,
        out_type=jax.ShapeDtypeStruct((B * S, 2 * D), jnp.float32),
        mesh=mesh,
        scratch_types=[
            pltpu.VMEM((B, SPW), jnp.int32),
            pltpu.VMEM((SPW, D), jnp.float32),
            pltpu.VMEM((2, BCHUNK, SPW, D), jnp.float32),
            pltpu.SemaphoreType.DMA,
            pltpu.SemaphoreType.DMA,
        ],
        compiler_params=pltpu.CompilerParams(use_tc_tiling_on_sc=False),
    )(_body)
    return f(x, pos, t3)[:, :D].reshape(B, S, D)


def kernel(x, table):
    return _run(x.reshape(-1), jnp.asarray(_POS), table)


# speedup vs baseline: 1.0891x; 1.0891x over previous
import functools

import numpy as np
import jax
import jax.numpy as jnp
from jax import lax
from jax.experimental import pallas as pl
from jax.experimental.pallas import tpu as pltpu, tpu_sc as plsc

VOCAB = 1000000
D = 64
B = 32
S = 2048

NC, NS, L = 2, 16, 16
NW = NC * NS
SPW = S // NW
BCHUNK = 4
NCHUNK = B // BCHUNK


def _positional_encoding_np(position, d_model):
    angle_rads = 1 / np.power(10000, 2 * (np.arange(d_model)[np.newaxis, :] // 2) / np.float32(d_model))
    angle_rads = np.arange(position)[:, np.newaxis] * angle_rads
    angle_rads[:, 0::2] = np.sin(angle_rads[:, 0::2])
    angle_rads[:, 1::2] = np.cos(angle_rads[:, 1::2])
    return angle_rads.astype(np.float32)


_POS = _positional_encoding_np(S, D)


def _body(x_hbm, pos_hbm, table_hbm, out_hbm, idx_v, pos_v, rows_v, gsem, wsem):
    wid = lax.axis_index("s") * NC + lax.axis_index("c")
    sbase = wid * SPW

    handles = [pltpu.async_copy(x_hbm.at[pl.ds(b * S + sbase, SPW)],
                                idx_v.at[b], gsem) for b in range(B)]
    pltpu.sync_copy(pos_hbm.at[pl.ds(sbase, SPW), :], pos_v)
    for h in handles:
        h.wait()

    def fire_gathers(cc, buf):
        return [pltpu.async_copy(table_hbm.at[idx_v.at[cc * BCHUNK + bb]],
                                 rows_v.at[buf, bb], gsem)
                for bb in range(BCHUNK)]

    gh = fire_gathers(0, 0)
    wh = []
    for cc in range(NCHUNK):
        cur = cc % 2
        for h in gh:
            h.wait()
        if cc + 1 < NCHUNK:
            for h in wh:
                h.wait()
            wh = []
            gh = fire_gathers(cc + 1, 1 - cur)

        def add_pos(j, carry):
            p = [pos_v[j, pl.ds(c * L, L)] for c in range(D // L)]
            for bb in range(BCHUNK):
                for c in range(D // L):
                    rows_v[cur, bb, j, pl.ds(c * L, L)] = (
                        rows_v[cur, bb, j, pl.ds(c * L, L)] + p[c])
            return carry
        lax.fori_loop(0, SPW, add_pos, 0)

        for h in wh:
            h.wait()
        wh = [pltpu.async_copy(rows_v.at[cur, bb],
                               out_hbm.at[pl.ds((cc * BCHUNK + bb) * S + sbase,
                                                SPW), :], wsem)
              for bb in range(BCHUNK)]
    for h in wh:
        h.wait()


@jax.jit
def _run(x, pos, table):
    t3 = jnp.concatenate([table, jnp.zeros((VOCAB, D), jnp.float32)], axis=1)
    mesh = plsc.VectorSubcoreMesh(core_axis_name="c", subcore_axis_name="s",
                                  num_cores=NC, num_subcores=NS)
    f = functools.partial(
        pl.kernel,
        out_type=jax.ShapeDtypeStruct((B * S, 2 * D), jnp.float32),
        mesh=mesh,
        scratch_types=[
            pltpu.VMEM((B, SPW), jnp.int32),
            pltpu.VMEM((SPW, D), jnp.float32),
            pltpu.VMEM((2, BCHUNK, SPW, 2 * D), jnp.float32),
            pltpu.SemaphoreType.DMA,
            pltpu.SemaphoreType.DMA,
        ],
        compiler_params=pltpu.CompilerParams(use_tc_tiling_on_sc=False),
    )(_body)
    return f(x, pos, t3)[:, :D].reshape(B, S, D)


def kernel(x, table):
    return _run(x.reshape(-1), jnp.asarray(_POS), table)
